# baseline (speedup 1.0000x reference)
import jax
import jax.numpy as jnp
from jax.experimental import pallas as pl
from jax.experimental.pallas import tpu as pltpu

BATCH_TILE = 512


def _nt_dot(a, b):
    return jax.lax.dot_general(
        a, b, (((1,), (1,)), ((), ())), preferred_element_type=jnp.float32)


def _fused_mlp_kernel(x_ref, w0_ref, m0_ref, b0_ref, w1_ref, m1_ref, b1_ref,
                      o_ref, wm0_ref, wm1_ref):
    D1 = w0_ref.shape[0]
    half = D1 // 2

    @pl.when(pl.program_id(0) == 0)
    def _prep_weights():
        w0b = w0_ref[:].astype(jnp.bfloat16)
        m0b = m0_ref[:].astype(jnp.bfloat16)
        wm0_ref[:] = w0b * m0b
        w1b = w1_ref[:].astype(jnp.bfloat16)
        m1b = m1_ref[:].astype(jnp.bfloat16)
        wm1_ref[:] = w1b * m1b

    xb = x_ref[:].astype(jnp.bfloat16)
    h0 = _nt_dot(xb, wm0_ref[:half, :])
    h0 = jnp.maximum(h0 + b0_ref[:, :half], 0.0).astype(jnp.bfloat16)
    h1 = _nt_dot(xb, wm0_ref[half:, :])
    o0 = _nt_dot(h0, wm1_ref[:, :half])
    h1 = jnp.maximum(h1 + b0_ref[:, half:], 0.0).astype(jnp.bfloat16)
    o1 = _nt_dot(h1, wm1_ref[:, half:])
    o_ref[:] = o0 + o1 + b1_ref[:]


def kernel(x, W0, b0, W1, b1, mask0, mask1):
    B, D0 = x.shape
    D1 = W0.shape[0]
    D2 = W1.shape[0]
    m0 = mask0.astype(jnp.int8)
    m1 = mask1.astype(jnp.int8)
    b0r = b0.reshape(1, D1)
    b1r = b1.reshape(1, D2)
    grid = (B // BATCH_TILE,)
    return pl.pallas_call(
        _fused_mlp_kernel,
        grid=grid,
        in_specs=[
            pl.BlockSpec((BATCH_TILE, D0), lambda i: (i, 0)),
            pl.BlockSpec((D1, D0), lambda i: (0, 0)),
            pl.BlockSpec((D1, D0), lambda i: (0, 0)),
            pl.BlockSpec((1, D1), lambda i: (0, 0)),
            pl.BlockSpec((D2, D1), lambda i: (0, 0)),
            pl.BlockSpec((D2, D1), lambda i: (0, 0)),
            pl.BlockSpec((1, D2), lambda i: (0, 0)),
        ],
        out_specs=pl.BlockSpec((BATCH_TILE, D2), lambda i: (i, 0)),
        out_shape=jax.ShapeDtypeStruct((B, D2), jnp.float32),
        scratch_shapes=[
            pltpu.VMEM((D1, D0), jnp.bfloat16),
            pltpu.VMEM((D2, D1), jnp.bfloat16),
        ],
    )(x, W0, m0, b0r, W1, m1, b1r)
